# baseline (speedup 1.0000x reference)
import functools

import jax
import jax.numpy as jnp
from jax import lax
from jax.experimental import pallas as pl
from jax.experimental.pallas import tpu as pltpu
from jax.experimental.pallas import tpu_sc as plsc

L = 16
NC = 2
NS = 16
NW = NC * NS


def _sc_combine(nm, b, f, t, bc):
    plane = bc * f
    nchunk = b // bc
    niter = -(-nchunk // NW)
    tpad = -(-t // L) * L
    nj = plane // L
    nz = nm * plane // L

    bf = b * f
    mesh = plsc.VectorSubcoreMesh(core_axis_name="c", subcore_axis_name="s")

    @functools.partial(
        pl.kernel,
        out_type=jax.ShapeDtypeStruct((nm * b * f,), jnp.float32),
        mesh=mesh,
        scratch_types=[
            pltpu.VMEM((nm * plane,), jnp.float32),
            pltpu.VMEM((nm * plane,), jnp.float32),
            pltpu.VMEM((nm * plane,), jnp.float32),
            pltpu.VMEM((tpad,), jnp.float32),
            pltpu.VMEM((tpad,), jnp.int32),
            pltpu.VMEM((tpad,), jnp.int32),
            pltpu.VMEM((tpad,), jnp.int32),
        ],
    )
    def combine(x1_hbm, x2_hbm, mult_hbm, o1_hbm, o2_hbm, oo_hbm, out_hbm,
                x1c, x2c, outc, multv, o1v, o2v, oov):
        wid = lax.axis_index("s") * NC + lax.axis_index("c")

        pltpu.sync_copy(mult_hbm, multv)
        pltpu.sync_copy(o1_hbm, o1v)
        pltpu.sync_copy(o2_hbm, o2v)
        pltpu.sync_copy(oo_hbm, oov)

        def extract(ref):
            vals = []
            for base in range(0, t, L):
                v = ref[pl.ds(base, L)]
                for lane in range(min(L, t - base)):
                    vals.append(v[lane])
            return vals

        mult_s = extract(multv)
        o1_s = extract(o1v)
        o2_s = extract(o2v)
        oo_s = extract(oov)

        zeros = jnp.zeros((L,), jnp.float32)

        def chunk_body(k, carry):
            c = wid + k * NW

            @pl.when(c < nchunk)
            def _():
                b0 = c * plane
                for m in range(nm):
                    pltpu.sync_copy(x1_hbm.at[pl.ds(m * bf + b0, plane)],
                                    x1c.at[pl.ds(m * plane, plane)])
                    pltpu.sync_copy(x2_hbm.at[pl.ds(m * bf + b0, plane)],
                                    x2c.at[pl.ds(m * plane, plane)])

                def zero_body(j, zc):
                    outc[pl.ds(j * L, L)] = zeros
                    return zc

                lax.fori_loop(0, nz, zero_body, 0, unroll=4)

                def j_body(j, jc):
                    off = j * L
                    for tt in range(t):
                        a = x1c[pl.ds(o1_s[tt] + off, L)]
                        bb = x2c[pl.ds(o2_s[tt] + off, L)]
                        plsc.addupdate(outc.at[pl.ds(oo_s[tt] + off, L)],
                                       a * bb * mult_s[tt])
                    return jc

                lax.fori_loop(0, nj, j_body, 0)

                for m in range(nm):
                    pltpu.sync_copy(outc.at[pl.ds(m * plane, plane)],
                                    out_hbm.at[pl.ds(m * bf + b0, plane)])

            return carry

        lax.fori_loop(0, niter, chunk_body, 0)

    return combine


def kernel(X1, X2, multipliers, m1_aligned, m2_aligned, mu):
    nm1, b, f = X1.shape
    nm2 = X2.shape[0]
    t = multipliers.shape[0]
    nmu = 9
    bc = 25

    plane = bc * f
    tpad = -(-t // L) * L
    pad = tpad - t

    x1f = X1.reshape(nm1 * b * f)
    x2f = X2.reshape(nm2 * b * f)
    multp = jnp.pad(multipliers, (0, pad))
    o1 = jnp.pad(m1_aligned.astype(jnp.int32) * plane, (0, pad))
    o2 = jnp.pad(m2_aligned.astype(jnp.int32) * plane, (0, pad))
    oo = jnp.pad(mu.astype(jnp.int32) * plane, (0, pad))

    fn = _sc_combine(nmu, b, f, t, bc)
    out = fn(x1f, x2f, multp, o1, o2, oo)
    return out.reshape(nmu, b, f)


# speedup vs baseline: 3.5153x; 3.5153x over previous
import functools

import jax
import jax.numpy as jnp
from jax import lax
from jax.experimental import pallas as pl
from jax.experimental.pallas import tpu as pltpu
from jax.experimental.pallas import tpu_sc as plsc

L = 16
NC = 2
NS = 16
NW = NC * NS


def _sc_combine(nm, b, f, t, bc):
    plane = bc * f
    nchunk = b // bc
    niter = -(-nchunk // NW)
    tpad = -(-t // L) * L
    nj = plane // L
    nz = nm * plane // L

    bf = b * f
    mesh = plsc.VectorSubcoreMesh(core_axis_name="c", subcore_axis_name="s")

    @functools.partial(
        pl.kernel,
        out_type=jax.ShapeDtypeStruct((nm * b * f,), jnp.float32),
        mesh=mesh,
        scratch_types=[
            pltpu.VMEM((nm * plane,), jnp.float32),
            pltpu.VMEM((nm * plane,), jnp.float32),
            pltpu.VMEM((nm * plane,), jnp.float32),
            pltpu.VMEM((tpad,), jnp.float32),
            pltpu.VMEM((tpad,), jnp.int32),
            pltpu.VMEM((tpad,), jnp.int32),
            pltpu.VMEM((tpad,), jnp.int32),
            pltpu.SemaphoreType.DMA,
            pltpu.SemaphoreType.DMA,
        ],
    )
    def combine(x1_hbm, x2_hbm, mult_hbm, o1_hbm, o2_hbm, oo_hbm, out_hbm,
                x1c, x2c, outc, multv, o1v, o2v, oov, sem_in, sem_out):
        wid = lax.axis_index("s") * NC + lax.axis_index("c")

        pltpu.sync_copy(mult_hbm, multv)
        pltpu.sync_copy(o1_hbm, o1v)
        pltpu.sync_copy(o2_hbm, o2v)
        pltpu.sync_copy(oo_hbm, oov)

        def extract(ref):
            vals = []
            for base in range(0, t, L):
                v = ref[pl.ds(base, L)]
                for lane in range(min(L, t - base)):
                    vals.append(v[lane])
            return vals

        mult_s = extract(multv)
        o1_s = extract(o1v)
        o2_s = extract(o2v)
        oo_s = extract(oov)

        zeros = jnp.zeros((L,), jnp.float32)

        def chunk_body(k, carry):
            c = wid + k * NW

            @pl.when(c < nchunk)
            def _():
                b0 = c * plane
                copies = []
                for m in range(nm):
                    copies.append(pltpu.async_copy(
                        x1_hbm.at[pl.ds(m * bf + b0, plane)],
                        x1c.at[pl.ds(m * plane, plane)], sem_in))
                    copies.append(pltpu.async_copy(
                        x2_hbm.at[pl.ds(m * bf + b0, plane)],
                        x2c.at[pl.ds(m * plane, plane)], sem_in))

                def zero_body(j, zc):
                    outc[pl.ds(j * L, L)] = zeros
                    return zc

                lax.fori_loop(0, nz, zero_body, 0, unroll=4)
                for cp in copies:
                    cp.wait()

                @plsc.parallel_loop(0, nj, 1, unroll=2)
                def j_body(j):
                    off = j * L
                    for tt in range(t):
                        a = x1c.at[
                            pl.ds(pl.multiple_of(o1_s[tt] + off, L), L)][...]
                        bb = x2c.at[
                            pl.ds(pl.multiple_of(o2_s[tt] + off, L), L)][...]
                        plsc.addupdate(
                            outc.at[pl.ds(pl.multiple_of(oo_s[tt] + off, L),
                                          L)],
                            a * bb * mult_s[tt])

                ocopies = []
                for m in range(nm):
                    ocopies.append(pltpu.async_copy(
                        outc.at[pl.ds(m * plane, plane)],
                        out_hbm.at[pl.ds(m * bf + b0, plane)], sem_out))
                for cp in ocopies:
                    cp.wait()

            return carry

        lax.fori_loop(0, niter, chunk_body, 0)

    return combine


def kernel(X1, X2, multipliers, m1_aligned, m2_aligned, mu):
    nm1, b, f = X1.shape
    nm2 = X2.shape[0]
    t = multipliers.shape[0]
    nmu = 9
    bc = 25

    plane = bc * f
    tpad = -(-t // L) * L
    pad = tpad - t

    x1f = X1.reshape(nm1 * b * f)
    x2f = X2.reshape(nm2 * b * f)
    multp = jnp.pad(multipliers, (0, pad))
    o1 = jnp.pad(m1_aligned.astype(jnp.int32) * plane, (0, pad))
    o2 = jnp.pad(m2_aligned.astype(jnp.int32) * plane, (0, pad))
    oo = jnp.pad(mu.astype(jnp.int32) * plane, (0, pad))

    fn = _sc_combine(nmu, b, f, t, bc)
    out = fn(x1f, x2f, multp, o1, o2, oo)
    return out.reshape(nmu, b, f)


# speedup vs baseline: 6.7373x; 1.9166x over previous
import functools

import jax
import jax.numpy as jnp
from jax import lax
from jax.experimental import pallas as pl
from jax.experimental.pallas import tpu as pltpu
from jax.experimental.pallas import tpu_sc as plsc

L = 16
NC = 2
NS = 16
NW = NC * NS


def _sc_combine(nm, b, bsc, f, t, bc):
    plane = bc * f
    nchunk = bsc // bc
    niter = -(-nchunk // NW)
    tpad = -(-t // L) * L
    nj = plane // L
    nz = nm * plane // L

    bf = b * f
    obf = bsc * f
    mesh = plsc.VectorSubcoreMesh(core_axis_name="c", subcore_axis_name="s")

    @functools.partial(
        pl.kernel,
        out_type=jax.ShapeDtypeStruct((nm * bsc * f,), jnp.float32),
        mesh=mesh,
        scratch_types=[
            pltpu.VMEM((nm * plane,), jnp.float32),
            pltpu.VMEM((nm * plane,), jnp.float32),
            pltpu.VMEM((nm * plane,), jnp.float32),
            pltpu.VMEM((tpad,), jnp.float32),
            pltpu.VMEM((tpad,), jnp.int32),
            pltpu.VMEM((tpad,), jnp.int32),
            pltpu.VMEM((tpad,), jnp.int32),
            pltpu.SemaphoreType.DMA,
            pltpu.SemaphoreType.DMA,
        ],
    )
    def combine(x1_hbm, x2_hbm, mult_hbm, o1_hbm, o2_hbm, oo_hbm, out_hbm,
                x1c, x2c, outc, multv, o1v, o2v, oov, sem_in, sem_out):
        wid = lax.axis_index("s") * NC + lax.axis_index("c")

        pltpu.sync_copy(mult_hbm, multv)
        pltpu.sync_copy(o1_hbm, o1v)
        pltpu.sync_copy(o2_hbm, o2v)
        pltpu.sync_copy(oo_hbm, oov)

        def extract(ref):
            vals = []
            for base in range(0, t, L):
                v = ref[pl.ds(base, L)]
                for lane in range(min(L, t - base)):
                    vals.append(v[lane])
            return vals

        mult_s = extract(multv)
        o1_s = extract(o1v)
        o2_s = extract(o2v)
        oo_s = extract(oov)

        zeros = jnp.zeros((L,), jnp.float32)

        def chunk_body(k, carry):
            c = wid + k * NW

            @pl.when(c < nchunk)
            def _():
                b0 = c * plane
                copies = []
                for m in range(nm):
                    copies.append(pltpu.async_copy(
                        x1_hbm.at[pl.ds(m * bf + b0, plane)],
                        x1c.at[pl.ds(m * plane, plane)], sem_in))
                    copies.append(pltpu.async_copy(
                        x2_hbm.at[pl.ds(m * bf + b0, plane)],
                        x2c.at[pl.ds(m * plane, plane)], sem_in))

                def zero_body(j, zc):
                    outc[pl.ds(j * L, L)] = zeros
                    return zc

                lax.fori_loop(0, nz, zero_body, 0, unroll=4)
                for cp in copies:
                    cp.wait()

                @plsc.parallel_loop(0, nj, 1, unroll=2)
                def j_body(j):
                    off = j * L
                    for tt in range(t):
                        a = x1c.at[
                            pl.ds(pl.multiple_of(o1_s[tt] + off, L), L)][...]
                        bb = x2c.at[
                            pl.ds(pl.multiple_of(o2_s[tt] + off, L), L)][...]
                        plsc.addupdate(
                            outc.at[pl.ds(pl.multiple_of(oo_s[tt] + off, L),
                                          L)],
                            a * bb * mult_s[tt])

                ocopies = []
                for m in range(nm):
                    ocopies.append(pltpu.async_copy(
                        outc.at[pl.ds(m * plane, plane)],
                        out_hbm.at[pl.ds(m * obf + b0, plane)], sem_out))
                for cp in ocopies:
                    cp.wait()

            return carry

        lax.fori_loop(0, niter, chunk_body, 0)

    return combine


def _tc_combine(nm, nmu, b, f, t, tb, off_blocks, nblocks):
    def body(m1_ref, m2_ref, mu_ref, mult_ref, x1_ref, x2_ref, out_ref):
        out_ref[...] = jnp.zeros_like(out_ref)
        for tt in range(t):
            a = x1_ref[pl.ds(m1_ref[tt], 1)]
            bb = x2_ref[pl.ds(m2_ref[tt], 1)]
            m = mu_ref[tt]
            out_ref[pl.ds(m, 1)] = (out_ref[pl.ds(m, 1)]
                                    + a * bb * mult_ref[tt])

    return pl.pallas_call(
        body,
        grid_spec=pltpu.PrefetchScalarGridSpec(
            num_scalar_prefetch=4,
            grid=(nblocks,),
            in_specs=[
                pl.BlockSpec((nm, tb, f), lambda i, *_: (0, i + off_blocks, 0)),
                pl.BlockSpec((nm, tb, f), lambda i, *_: (0, i + off_blocks, 0)),
            ],
            out_specs=pl.BlockSpec((nmu, tb, f), lambda i, *_: (0, i, 0)),
        ),
        out_shape=jax.ShapeDtypeStruct((nmu, nblocks * tb, f), jnp.float32),
    )


def kernel(X1, X2, multipliers, m1_aligned, m2_aligned, mu):
    nm1, b, f = X1.shape
    nm2 = X2.shape[0]
    t = multipliers.shape[0]
    nmu = 9
    bc = 25
    bsc = 4000
    tb = 1000

    plane = bc * f
    tpad = -(-t // L) * L
    pad = tpad - t

    x1f = X1.reshape(nm1 * b * f)
    x2f = X2.reshape(nm2 * b * f)
    multp = jnp.pad(multipliers, (0, pad))
    m1i = m1_aligned.astype(jnp.int32)
    m2i = m2_aligned.astype(jnp.int32)
    mui = mu.astype(jnp.int32)
    o1 = jnp.pad(m1i * plane, (0, pad))
    o2 = jnp.pad(m2i * plane, (0, pad))
    oo = jnp.pad(mui * plane, (0, pad))

    sc_fn = _sc_combine(nmu, b, bsc, f, t, bc)
    sc_out = sc_fn(x1f, x2f, multp, o1, o2, oo)

    if bsc < b:
        tc_fn = _tc_combine(nm1, nmu, b, f, t, tb, bsc // tb, (b - bsc) // tb)
        tc_out = tc_fn(m1i, m2i, mui, multipliers, X1, X2)
        return jnp.concatenate(
            [sc_out.reshape(nmu, bsc, f), tc_out], axis=1)
    return sc_out.reshape(nmu, b, f)


# speedup vs baseline: 9.6106x; 1.4265x over previous
import functools

import jax
import jax.numpy as jnp
from jax import lax
from jax.experimental import pallas as pl
from jax.experimental.pallas import tpu as pltpu
from jax.experimental.pallas import tpu_sc as plsc

L = 16
NC = 2
NS = 16
NW = NC * NS


def _sc_combine(nm, b, bsc, f, t, bc):
    plane = bc * f
    nchunk = bsc // bc
    niter = -(-nchunk // NW)
    tpad = -(-t // L) * L
    nj = plane // L
    nz = nm * plane // L

    bf = b * f
    obf = b * f
    mesh = plsc.VectorSubcoreMesh(core_axis_name="c", subcore_axis_name="s")

    @functools.partial(
        pl.kernel,
        out_type=jax.ShapeDtypeStruct((nm * b * f,), jnp.float32),
        mesh=mesh,
        scratch_types=[
            pltpu.VMEM((nm * plane,), jnp.float32),
            pltpu.VMEM((nm * plane,), jnp.float32),
            pltpu.VMEM((nm * plane,), jnp.float32),
            pltpu.VMEM((tpad,), jnp.float32),
            pltpu.VMEM((tpad,), jnp.int32),
            pltpu.VMEM((tpad,), jnp.int32),
            pltpu.VMEM((tpad,), jnp.int32),
            pltpu.SemaphoreType.DMA,
            pltpu.SemaphoreType.DMA,
        ],
    )
    def combine(x1_hbm, x2_hbm, mult_hbm, o1_hbm, o2_hbm, oo_hbm, out_hbm,
                x1c, x2c, outc, multv, o1v, o2v, oov, sem_in, sem_out):
        wid = lax.axis_index("s") * NC + lax.axis_index("c")

        pltpu.sync_copy(mult_hbm, multv)
        pltpu.sync_copy(o1_hbm, o1v)
        pltpu.sync_copy(o2_hbm, o2v)
        pltpu.sync_copy(oo_hbm, oov)

        def extract(ref):
            vals = []
            for base in range(0, t, L):
                v = ref[pl.ds(base, L)]
                for lane in range(min(L, t - base)):
                    vals.append(v[lane])
            return vals

        mult_s = extract(multv)
        o1_s = extract(o1v)
        o2_s = extract(o2v)
        oo_s = extract(oov)

        zeros = jnp.zeros((L,), jnp.float32)

        def chunk_body(k, carry):
            c = wid + k * NW

            @pl.when(c < nchunk)
            def _():
                b0 = c * plane
                copies = []
                for m in range(nm):
                    copies.append(pltpu.async_copy(
                        x1_hbm.at[pl.ds(m * bf + b0, plane)],
                        x1c.at[pl.ds(m * plane, plane)], sem_in))
                    copies.append(pltpu.async_copy(
                        x2_hbm.at[pl.ds(m * bf + b0, plane)],
                        x2c.at[pl.ds(m * plane, plane)], sem_in))

                def zero_body(j, zc):
                    outc[pl.ds(j * L, L)] = zeros
                    return zc

                lax.fori_loop(0, nz, zero_body, 0, unroll=4)
                for cp in copies:
                    cp.wait()

                @plsc.parallel_loop(0, nj, 1, unroll=2)
                def j_body(j):
                    off = j * L
                    for tt in range(t):
                        a = x1c.at[
                            pl.ds(pl.multiple_of(o1_s[tt] + off, L), L)][...]
                        bb = x2c.at[
                            pl.ds(pl.multiple_of(o2_s[tt] + off, L), L)][...]
                        plsc.addupdate(
                            outc.at[pl.ds(pl.multiple_of(oo_s[tt] + off, L),
                                          L)],
                            a * bb * mult_s[tt])

                ocopies = []
                for m in range(nm):
                    ocopies.append(pltpu.async_copy(
                        outc.at[pl.ds(m * plane, plane)],
                        out_hbm.at[pl.ds(m * obf + b0, plane)], sem_out))
                for cp in ocopies:
                    cp.wait()

            return carry

        lax.fori_loop(0, niter, chunk_body, 0)

    return combine


def _tc_combine(nm, nmu, b, f, t, tb, off_blocks, nblocks):
    def body(m1_ref, m2_ref, mu_ref, mult_ref, x1_ref, x2_ref, sc_ref,
             out_ref):
        del sc_ref
        out_ref[...] = jnp.zeros_like(out_ref)
        for tt in range(t):
            a = x1_ref[pl.ds(m1_ref[tt], 1)]
            bb = x2_ref[pl.ds(m2_ref[tt], 1)]
            m = mu_ref[tt]
            out_ref[pl.ds(m, 1)] = (out_ref[pl.ds(m, 1)]
                                    + a * bb * mult_ref[tt])

    return pl.pallas_call(
        body,
        grid_spec=pltpu.PrefetchScalarGridSpec(
            num_scalar_prefetch=4,
            grid=(nblocks,),
            in_specs=[
                pl.BlockSpec((nm, tb, f), lambda i, *_: (0, i + off_blocks, 0)),
                pl.BlockSpec((nm, tb, f), lambda i, *_: (0, i + off_blocks, 0)),
                pl.BlockSpec(memory_space=pl.ANY),
            ],
            out_specs=pl.BlockSpec((nmu, tb, f),
                                   lambda i, *_: (0, i + off_blocks, 0)),
        ),
        out_shape=jax.ShapeDtypeStruct((nmu, b, f), jnp.float32),
        input_output_aliases={6: 0},
    )


def kernel(X1, X2, multipliers, m1_aligned, m2_aligned, mu):
    nm1, b, f = X1.shape
    nm2 = X2.shape[0]
    t = multipliers.shape[0]
    nmu = 9
    bc = 25
    bsc = 1000
    tb = 1000

    plane = bc * f
    tpad = -(-t // L) * L
    pad = tpad - t

    x1f = X1.reshape(nm1 * b * f)
    x2f = X2.reshape(nm2 * b * f)
    multp = jnp.pad(multipliers, (0, pad))
    m1i = m1_aligned.astype(jnp.int32)
    m2i = m2_aligned.astype(jnp.int32)
    mui = mu.astype(jnp.int32)
    o1 = jnp.pad(m1i * plane, (0, pad))
    o2 = jnp.pad(m2i * plane, (0, pad))
    oo = jnp.pad(mui * plane, (0, pad))

    sc_fn = _sc_combine(nmu, b, bsc, f, t, bc)
    sc_out = sc_fn(x1f, x2f, multp, o1, o2, oo)

    if bsc < b:
        tc_fn = _tc_combine(nm1, nmu, b, f, t, tb, bsc // tb, (b - bsc) // tb)
        return tc_fn(m1i, m2i, mui, multipliers, X1, X2,
                     sc_out.reshape(nmu, b, f))
    return sc_out.reshape(nmu, b, f)


# speedup vs baseline: 12.3651x; 1.2866x over previous
import functools

import jax
import jax.numpy as jnp
from jax import lax
from jax.experimental import pallas as pl
from jax.experimental.pallas import tpu as pltpu
from jax.experimental.pallas import tpu_sc as plsc

L = 16
NC = 2
NS = 16
NW = NC * NS


def _sc_combine(nm, b, bsc, f, t, bc):
    plane = bc * f
    nchunk = bsc // bc
    niter = -(-nchunk // NW)
    tpad = -(-t // L) * L
    nj = plane // L
    nz = nm * plane // L

    bf = b * f
    obf = b * f
    mesh = plsc.VectorSubcoreMesh(core_axis_name="c", subcore_axis_name="s")

    @functools.partial(
        pl.kernel,
        out_type=jax.ShapeDtypeStruct((nm * b * f,), jnp.float32),
        mesh=mesh,
        scratch_types=[
            pltpu.VMEM((nm * plane,), jnp.float32),
            pltpu.VMEM((nm * plane,), jnp.float32),
            pltpu.VMEM((nm * plane,), jnp.float32),
            pltpu.VMEM((tpad,), jnp.float32),
            pltpu.VMEM((tpad,), jnp.int32),
            pltpu.VMEM((tpad,), jnp.int32),
            pltpu.VMEM((tpad,), jnp.int32),
            pltpu.SemaphoreType.DMA,
            pltpu.SemaphoreType.DMA,
        ],
    )
    def combine(x1_hbm, x2_hbm, mult_hbm, o1_hbm, o2_hbm, oo_hbm, out_hbm,
                x1c, x2c, outc, multv, o1v, o2v, oov, sem_in, sem_out):
        wid = lax.axis_index("s") * NC + lax.axis_index("c")

        pltpu.sync_copy(mult_hbm, multv)
        pltpu.sync_copy(o1_hbm, o1v)
        pltpu.sync_copy(o2_hbm, o2v)
        pltpu.sync_copy(oo_hbm, oov)

        def extract(ref):
            vals = []
            for base in range(0, t, L):
                v = ref[pl.ds(base, L)]
                for lane in range(min(L, t - base)):
                    vals.append(v[lane])
            return vals

        mult_s = extract(multv)
        o1_s = extract(o1v)
        o2_s = extract(o2v)
        oo_s = extract(oov)

        zeros = jnp.zeros((L,), jnp.float32)

        def chunk_body(k, carry):
            c = wid + k * NW

            @pl.when(c < nchunk)
            def _():
                b0 = c * plane
                copies = []
                for m in range(nm):
                    copies.append(pltpu.async_copy(
                        x1_hbm.at[pl.ds(m * bf + b0, plane)],
                        x1c.at[pl.ds(m * plane, plane)], sem_in))
                    copies.append(pltpu.async_copy(
                        x2_hbm.at[pl.ds(m * bf + b0, plane)],
                        x2c.at[pl.ds(m * plane, plane)], sem_in))

                def zero_body(j, zc):
                    outc[pl.ds(j * L, L)] = zeros
                    return zc

                lax.fori_loop(0, nz, zero_body, 0, unroll=4)
                for cp in copies:
                    cp.wait()

                @plsc.parallel_loop(0, nj, 1, unroll=4)
                def j_body(j):
                    off = j * L
                    for tt in range(t):
                        a = x1c.at[
                            pl.ds(pl.multiple_of(o1_s[tt] + off, L), L)][...]
                        bb = x2c.at[
                            pl.ds(pl.multiple_of(o2_s[tt] + off, L), L)][...]
                        plsc.addupdate(
                            outc.at[pl.ds(pl.multiple_of(oo_s[tt] + off, L),
                                          L)],
                            a * bb * mult_s[tt])

                ocopies = []
                for m in range(nm):
                    ocopies.append(pltpu.async_copy(
                        outc.at[pl.ds(m * plane, plane)],
                        out_hbm.at[pl.ds(m * obf + b0, plane)], sem_out))
                for cp in ocopies:
                    cp.wait()

            return carry

        lax.fori_loop(0, niter, chunk_body, 0)

    return combine


def _tc_combine(nm, nmu, b, f, t, tb, off_blocks, nblocks):
    def body(m1_ref, m2_ref, mu_ref, mult_ref, x1_ref, x2_ref, sc_ref,
             out_ref):
        del sc_ref
        out_ref[...] = jnp.zeros_like(out_ref)
        for tt in range(t):
            a = x1_ref[pl.ds(m1_ref[tt], 1)]
            bb = x2_ref[pl.ds(m2_ref[tt], 1)]
            m = mu_ref[tt]
            out_ref[pl.ds(m, 1)] = (out_ref[pl.ds(m, 1)]
                                    + a * bb * mult_ref[tt])

    return pl.pallas_call(
        body,
        grid_spec=pltpu.PrefetchScalarGridSpec(
            num_scalar_prefetch=4,
            grid=(nblocks,),
            in_specs=[
                pl.BlockSpec((nm, tb, f), lambda i, *_: (0, i + off_blocks, 0)),
                pl.BlockSpec((nm, tb, f), lambda i, *_: (0, i + off_blocks, 0)),
                pl.BlockSpec(memory_space=pl.ANY),
            ],
            out_specs=pl.BlockSpec((nmu, tb, f),
                                   lambda i, *_: (0, i + off_blocks, 0)),
        ),
        out_shape=jax.ShapeDtypeStruct((nmu, b, f), jnp.float32),
        input_output_aliases={6: 0},
    )


def kernel(X1, X2, multipliers, m1_aligned, m2_aligned, mu):
    nm1, b, f = X1.shape
    nm2 = X2.shape[0]
    t = multipliers.shape[0]
    nmu = 9
    bc = 25
    bsc = 800
    tb = 400

    plane = bc * f
    tpad = -(-t // L) * L
    pad = tpad - t

    x1f = X1.reshape(nm1 * b * f)
    x2f = X2.reshape(nm2 * b * f)
    multp = jnp.pad(multipliers, (0, pad))
    m1i = m1_aligned.astype(jnp.int32)
    m2i = m2_aligned.astype(jnp.int32)
    mui = mu.astype(jnp.int32)
    o1 = jnp.pad(m1i * plane, (0, pad))
    o2 = jnp.pad(m2i * plane, (0, pad))
    oo = jnp.pad(mui * plane, (0, pad))

    sc_fn = _sc_combine(nmu, b, bsc, f, t, bc)
    sc_out = sc_fn(x1f, x2f, multp, o1, o2, oo)

    if bsc < b:
        tc_fn = _tc_combine(nm1, nmu, b, f, t, tb, bsc // tb, (b - bsc) // tb)
        return tc_fn(m1i, m2i, mui, multipliers, X1, X2,
                     sc_out.reshape(nmu, b, f))
    return sc_out.reshape(nmu, b, f)
